# baseline (speedup 1.0000x reference)
import functools

import jax
import jax.numpy as jnp
from jax import lax
from jax.experimental import pallas as pl
from jax.experimental.pallas import tpu as pltpu
from jax.experimental.pallas import tpu_sc as plsc

_N = 10000
_E = 320000
_D_IN = 128
_D_HID = 128
_D_OUT = 64

_NC = 2
_NS = 16
_NW = _NC * _NS
_EPW = _E // _NW
_NBUF = 4
_NSET = 8
_RPT = 624
_TAIL = _N - _NS * _RPT
_TOFF = _NS * _RPT

_BLK = 2000
_GRID = _N // _BLK


def _make_sc_agg(D, with_count):
  ch = 40 if with_count else 80
  nchunk = _EPW // ch
  out_type = [jax.ShapeDtypeStruct((_NC, _N, D), jnp.float32)]
  scratch = (
      [pltpu.VMEM((ch,), jnp.int32)] * (2 * _NSET)
      + [pltpu.VMEM((ch, D), jnp.float32)] * _NBUF
      + [pltpu.VMEM_SHARED((_N, D), jnp.float32)]
      + [pltpu.SemaphoreType.DMA] * (_NSET + 2 * _NBUF)
  )
  if with_count:
    out_type.append(jax.ShapeDtypeStruct((_NW, 1, _N), jnp.float32))
    scratch.append(pltpu.VMEM((1, _N), jnp.float32))

  mesh = plsc.VectorSubcoreMesh(
      core_axis_name="c", subcore_axis_name="s",
      num_cores=_NC, num_subcores=_NS)

  def body(*refs):
    if with_count:
      (p_hbm, src_hbm, dst_hbm, z_hbm, zc_hbm, out_hbm, cnt_hbm) = refs[:7]
      rest = refs[7:]
      cnt = rest[-1]
      rest = rest[:-1]
    else:
      (p_hbm, src_hbm, dst_hbm, z_hbm, out_hbm) = refs[:5]
      rest = refs[5:]
    idx_refs = rest[:2 * _NSET]
    row_refs = rest[2 * _NSET:2 * _NSET + _NBUF]
    acc = rest[2 * _NSET + _NBUF]
    sems = rest[2 * _NSET + _NBUF + 1:]
    isems = sems[:_NSET]
    gsems = sems[_NSET:_NSET + _NBUF]
    ssems = sems[_NSET + _NBUF:]
    sets = [(idx_refs[2 * k], idx_refs[2 * k + 1], isems[k])
            for k in range(_NSET)]
    bufs = [(row_refs[b], gsems[b], ssems[b]) for b in range(_NBUF)]

    cid = lax.axis_index("c")
    sid = lax.axis_index("s")
    wid = cid * _NS + sid
    row0 = sid * _RPT

    if with_count:
      pltpu.sync_copy(zc_hbm, cnt)
    pltpu.sync_copy(z_hbm, acc.at[pl.ds(row0, _RPT)])

    @pl.when(sid == 0)
    def _zero_tail():
      pltpu.sync_copy(z_hbm.at[pl.ds(0, _TAIL)], acc.at[pl.ds(_TOFF, _TAIL)])

    plsc.subcore_barrier()

    def _idx(i, st):
      si, di, sem = st
      base = pl.multiple_of(wid * _EPW + i * ch, 8)
      pltpu.async_copy(src_hbm.at[pl.ds(base, ch)], si, sem)
      pltpu.async_copy(dst_hbm.at[pl.ds(base, ch)], di, sem)

    def _iwait(st):
      si, di, sem = st
      pltpu.make_async_copy(src_hbm.at[pl.ds(0, ch)], si, sem).wait()
      pltpu.make_async_copy(src_hbm.at[pl.ds(0, ch)], di, sem).wait()

    def _gather(st, bf):
      pltpu.async_copy(p_hbm.at[st[0]], bf[0], bf[1])

    def _gwait(st, bf):
      pltpu.make_async_copy(p_hbm.at[st[0]], bf[0], bf[1]).wait()

    def _scatter(st, bf):
      pltpu.make_async_copy(bf[0], acc.at[st[1]], bf[2]).start(add=True)

    def _swait(st, bf):
      pltpu.make_async_copy(bf[0], acc.at[st[1]], bf[2]).wait()

    def _counts(st):
      if with_count:
        zero16 = jnp.zeros((16,), jnp.int32)
        one16 = jnp.ones((16,), jnp.float32)
        for g in range(ch // 16):
          iv = st[1][pl.ds(g * 16, 16)]
          plsc.addupdate_scatter(cnt, [zero16, iv], one16)
        rem = ch % 16
        if rem:
          iv = st[1][pl.ds(ch - 16, 16)]
          msk = lax.iota(jnp.int32, 16) >= (16 - rem)
          plsc.addupdate_scatter(cnt, [zero16, iv], one16, mask=msk)

    def _when(pred, fn):
      if isinstance(pred, bool):
        if pred:
          fn()
      else:
        pl.when(pred)(fn)

    def slot(c, i):
      st, bf = sets[i], bufs[i % _NBUF]
      pst, pbf = sets[(i - 1) % _NSET], bufs[(i - 1) % _NBUF]

      _when(c >= _NBUF, lambda: _swait(st, bf))
      _when(c + 3 < nchunk, lambda: _idx(c + 3, sets[(i + 3) % _NSET]))
      _iwait(st)
      _gather(st, bf)
      _counts(st)

      def _finish_prev():
        _gwait(pst, pbf)
        _scatter(pst, pbf)
      _when(c >= 1, _finish_prev)

    for i in range(3):
      _idx(i, sets[i])

    nmain = (nchunk // _NSET) * _NSET

    def step(k, carry):
      for i in range(_NSET):
        slot(k * _NSET + i, i)
      return carry

    lax.fori_loop(0, nchunk // _NSET, step, 0)
    for c in range(nmain, nchunk):
      slot(c, c % _NSET)

    last = nchunk - 1
    _gwait(sets[last % _NSET], bufs[last % _NBUF])
    _scatter(sets[last % _NSET], bufs[last % _NBUF])
    for b in range(_NBUF):
      _swait(sets[0], bufs[b])
    plsc.subcore_barrier()

    pltpu.sync_copy(acc.at[pl.ds(row0, _RPT)],
                    out_hbm.at[cid, pl.ds(row0, _RPT)])
    if with_count:
      pltpu.sync_copy(cnt, cnt_hbm.at[wid])

    @pl.when(sid == 0)
    def _write_tail():
      pltpu.sync_copy(acc.at[pl.ds(_TOFF, _TAIL)],
                      out_hbm.at[cid, pl.ds(_TOFF, _TAIL)])

  kw = {}
  if with_count:
    kw["compiler_params"] = pltpu.CompilerParams(needs_layout_passes=False)
  return functools.partial(
      pl.kernel,
      out_type=out_type if with_count else out_type[0],
      mesh=mesh,
      scratch_types=scratch,
      **kw,
  )(body)


@functools.lru_cache(maxsize=None)
def _sc_agg(D, with_count):
  return _make_sc_agg(D, with_count)


def _tc1_body(x_ref, wl_ref, wr_ref, bl_ref, p_ref, r_ref):
  xv = x_ref[...]
  p_ref[...] = lax.dot_general(xv, wl_ref[...], (((1,), (1,)), ((), ())),
                               preferred_element_type=jnp.float32)
  r_ref[...] = lax.dot_general(xv, wr_ref[...], (((1,), (1,)), ((), ())),
                               preferred_element_type=jnp.float32) + bl_ref[...]


_tc1 = pl.pallas_call(
    _tc1_body,
    grid=(_GRID,),
    in_specs=[
        pl.BlockSpec((_BLK, _D_IN), lambda i: (i, 0)),
        pl.BlockSpec((_D_HID, _D_IN), lambda i: (0, 0)),
        pl.BlockSpec((_D_HID, _D_IN), lambda i: (0, 0)),
        pl.BlockSpec((1, _D_HID), lambda i: (0, 0)),
    ],
    out_specs=[
        pl.BlockSpec((_BLK, _D_HID), lambda i: (i, 0)),
        pl.BlockSpec((_BLK, _D_HID), lambda i: (i, 0)),
    ],
    out_shape=[
        jax.ShapeDtypeStruct((_N, _D_HID), jnp.float32),
        jax.ShapeDtypeStruct((_N, _D_HID), jnp.float32),
    ],
)


def _inv_bcast(cnt):
  total = jnp.sum(cnt, axis=0, keepdims=True)
  inv = 1.0 / jnp.maximum(total, 1.0)
  ones = jnp.ones((1, _D_HID), jnp.float32)
  return lax.dot_general(inv, ones, (((0,), (0,)), ((), ())),
                         preferred_element_type=jnp.float32)


def _tc2_body(agg_ref, cnt_ref, r1_ref, h_ref):
  invb = _inv_bcast(cnt_ref[...])
  mean = (agg_ref[0] + agg_ref[1]) * invb
  h_ref[...] = jnp.maximum(mean + r1_ref[...], 0.0)


_tc2 = pl.pallas_call(
    _tc2_body,
    in_specs=[
        pl.BlockSpec((_NC, _N, _D_HID), lambda: (0, 0, 0)),
        pl.BlockSpec((_NW, _N), lambda: (0, 0)),
        pl.BlockSpec((_N, _D_HID), lambda: (0, 0)),
    ],
    out_specs=pl.BlockSpec((_N, _D_HID), lambda: (0, 0)),
    out_shape=jax.ShapeDtypeStruct((_N, _D_HID), jnp.float32),
)


def _tc3_body(agg_ref, cnt_ref, h_ref, wl_ref, wr_ref, bl_ref, out_ref):
  invb = _inv_bcast(cnt_ref[...])
  mean = (agg_ref[0] + agg_ref[1]) * invb
  out_ref[...] = (
      lax.dot_general(mean, wl_ref[...], (((1,), (1,)), ((), ())),
                      preferred_element_type=jnp.float32)
      + lax.dot_general(h_ref[...], wr_ref[...], (((1,), (1,)), ((), ())),
                        preferred_element_type=jnp.float32)
      + bl_ref[...])


_tc3 = pl.pallas_call(
    _tc3_body,
    in_specs=[
        pl.BlockSpec((_NC, _N, _D_HID), lambda: (0, 0, 0)),
        pl.BlockSpec((_NW, _N), lambda: (0, 0)),
        pl.BlockSpec((_N, _D_HID), lambda: (0, 0)),
        pl.BlockSpec((_D_OUT, _D_HID), lambda: (0, 0)),
        pl.BlockSpec((_D_OUT, _D_HID), lambda: (0, 0)),
        pl.BlockSpec((1, _D_OUT), lambda: (0, 0)),
    ],
    out_specs=pl.BlockSpec((_N, _D_OUT), lambda: (0, 0)),
    out_shape=jax.ShapeDtypeStruct((_N, _D_OUT), jnp.float32),
)


def kernel(x, edge_index, Wl1, bl1, Wr1, Wl2, bl2, Wr2):
  src = edge_index[0]
  dst = edge_index[1]

  z128 = jnp.zeros((_RPT, _D_HID), jnp.float32)
  zc = jnp.zeros((1, _N), jnp.float32)

  p1, r1 = _tc1(x, Wl1, Wr1, bl1.reshape(1, -1))
  agg1, cnt = _sc_agg(_D_HID, True)(p1, src, dst, z128, zc)
  cnt2 = cnt.reshape(_NW, _N)
  h = _tc2(agg1, cnt2, r1)
  agg2 = _sc_agg(_D_HID, False)(h, src, dst, z128)
  return _tc3(agg2, cnt2, h, Wl2, Wr2, bl2.reshape(1, -1))


# speedup vs baseline: 1.0666x; 1.0666x over previous
import functools

import jax
import jax.numpy as jnp
from jax import lax
from jax.experimental import pallas as pl
from jax.experimental.pallas import tpu as pltpu
from jax.experimental.pallas import tpu_sc as plsc

_N = 10000
_E = 320000
_D_IN = 128
_D_HID = 128
_D_OUT = 64

_NC = 2
_NS = 16
_NW = _NC * _NS
_EPW = _E // _NW
_CH = 80
_RPT = 624
_TAIL = _N - _NS * _RPT
_TOFF = _NS * _RPT

_BLK = 2000
_GRID = _N // _BLK


def _make_sc_agg(D, with_count):
  ch = _CH
  nbuf = 2 if with_count else 4
  nset = 4 if with_count else 8
  pf = nset - nbuf
  nchunk = _EPW // ch
  out_type = [jax.ShapeDtypeStruct((_NC, _N, D), jnp.float32)]
  scratch = (
      [pltpu.VMEM((ch,), jnp.int32)] * (2 * nset)
      + [pltpu.VMEM((ch, D), jnp.float32)] * nbuf
      + [pltpu.VMEM_SHARED((_N, D), jnp.float32)]
      + [pltpu.SemaphoreType.DMA] * (nset + 2 * nbuf)
  )
  if with_count:
    out_type.append(jax.ShapeDtypeStruct((_NW, 1, _N), jnp.float32))
    scratch.append(pltpu.VMEM((1, _N), jnp.float32))

  mesh = plsc.VectorSubcoreMesh(
      core_axis_name="c", subcore_axis_name="s",
      num_cores=_NC, num_subcores=_NS)

  def body(*refs):
    if with_count:
      (p_hbm, src_hbm, dst_hbm, z_hbm, zc_hbm, out_hbm, cnt_hbm) = refs[:7]
      rest = refs[7:]
      cnt = rest[-1]
      rest = rest[:-1]
    else:
      (p_hbm, src_hbm, dst_hbm, z_hbm, out_hbm) = refs[:5]
      rest = refs[5:]
    idx_refs = rest[:2 * nset]
    row_refs = rest[2 * nset:2 * nset + nbuf]
    acc = rest[2 * nset + nbuf]
    sems = rest[2 * nset + nbuf + 1:]
    isems = sems[:nset]
    gsems = sems[nset:nset + nbuf]
    ssems = sems[nset + nbuf:]
    sets = [(idx_refs[2 * k], idx_refs[2 * k + 1], isems[k])
            for k in range(nset)]
    bufs = [(row_refs[b], gsems[b], ssems[b]) for b in range(nbuf)]

    cid = lax.axis_index("c")
    sid = lax.axis_index("s")
    wid = cid * _NS + sid
    row0 = sid * _RPT

    if with_count:
      pltpu.sync_copy(zc_hbm, cnt)
    pltpu.sync_copy(z_hbm, acc.at[pl.ds(row0, _RPT)])

    @pl.when(sid == 0)
    def _zero_tail():
      pltpu.sync_copy(z_hbm.at[pl.ds(0, _TAIL)], acc.at[pl.ds(_TOFF, _TAIL)])

    plsc.subcore_barrier()

    def _idx(i, st):
      si, di, sem = st
      base = pl.multiple_of(wid * _EPW + i * ch, 8)
      pltpu.async_copy(src_hbm.at[pl.ds(base, ch)], si, sem)
      pltpu.async_copy(dst_hbm.at[pl.ds(base, ch)], di, sem)

    def _iwait(st):
      si, di, sem = st
      pltpu.make_async_copy(src_hbm.at[pl.ds(0, ch)], si, sem).wait()
      pltpu.make_async_copy(src_hbm.at[pl.ds(0, ch)], di, sem).wait()

    def _gather(st, bf):
      pltpu.async_copy(p_hbm.at[st[0]], bf[0], bf[1])

    def _gwait(st, bf):
      pltpu.make_async_copy(p_hbm.at[st[0]], bf[0], bf[1]).wait()

    def _scatter(st, bf):
      pltpu.make_async_copy(bf[0], acc.at[st[1]], bf[2]).start(add=True)

    def _swait(st, bf):
      pltpu.make_async_copy(bf[0], acc.at[st[1]], bf[2]).wait()

    def _counts(st):
      if with_count:
        zero16 = jnp.zeros((16,), jnp.int32)
        one16 = jnp.ones((16,), jnp.float32)
        for g in range(ch // 16):
          iv = st[1][pl.ds(g * 16, 16)]
          plsc.addupdate_scatter(cnt, [zero16, iv], one16)
        rem = ch % 16
        if rem:
          iv = st[1][pl.ds(ch - 16, 16)]
          msk = lax.iota(jnp.int32, 16) >= (16 - rem)
          plsc.addupdate_scatter(cnt, [zero16, iv], one16, mask=msk)

    def _when(pred, fn):
      if isinstance(pred, bool):
        if pred:
          fn()
      else:
        pl.when(pred)(fn)

    def slot(c, i):
      st, bf = sets[i], bufs[i % nbuf]
      pst, pbf = sets[(i - 1) % nset], bufs[(i - 1) % nbuf]

      _when(c >= nbuf, lambda: _swait(st, bf))
      _when(c + pf < nchunk, lambda: _idx(c + pf, sets[(i + pf) % nset]))
      _iwait(st)
      _gather(st, bf)
      _counts(st)

      def _finish_prev():
        _gwait(pst, pbf)
        _scatter(pst, pbf)
      _when(c >= 1, _finish_prev)

    for i in range(pf):
      _idx(i, sets[i])

    nmain = (nchunk // nset) * nset

    def step(k, carry):
      for i in range(nset):
        slot(k * nset + i, i)
      return carry

    lax.fori_loop(0, nchunk // nset, step, 0)
    for c in range(nmain, nchunk):
      slot(c, c % nset)

    last = nchunk - 1
    _gwait(sets[last % nset], bufs[last % nbuf])
    _scatter(sets[last % nset], bufs[last % nbuf])
    for b in range(nbuf):
      _swait(sets[0], bufs[b])
    plsc.subcore_barrier()

    pltpu.sync_copy(acc.at[pl.ds(row0, _RPT)],
                    out_hbm.at[cid, pl.ds(row0, _RPT)])
    if with_count:
      pltpu.sync_copy(cnt, cnt_hbm.at[wid])

    @pl.when(sid == 0)
    def _write_tail():
      pltpu.sync_copy(acc.at[pl.ds(_TOFF, _TAIL)],
                      out_hbm.at[cid, pl.ds(_TOFF, _TAIL)])

  kw = {}
  if with_count:
    kw["compiler_params"] = pltpu.CompilerParams(needs_layout_passes=False)
  return functools.partial(
      pl.kernel,
      out_type=out_type if with_count else out_type[0],
      mesh=mesh,
      scratch_types=scratch,
      **kw,
  )(body)


@functools.lru_cache(maxsize=None)
def _sc_agg(D, with_count):
  return _make_sc_agg(D, with_count)


def _tc1_body(x_ref, wl_ref, wr_ref, bl_ref, p_ref, r_ref):
  xv = x_ref[...]
  p_ref[...] = lax.dot_general(xv, wl_ref[...], (((1,), (1,)), ((), ())),
                               preferred_element_type=jnp.float32)
  r_ref[...] = lax.dot_general(xv, wr_ref[...], (((1,), (1,)), ((), ())),
                               preferred_element_type=jnp.float32) + bl_ref[...]


_tc1 = pl.pallas_call(
    _tc1_body,
    grid=(_GRID,),
    in_specs=[
        pl.BlockSpec((_BLK, _D_IN), lambda i: (i, 0)),
        pl.BlockSpec((_D_HID, _D_IN), lambda i: (0, 0)),
        pl.BlockSpec((_D_HID, _D_IN), lambda i: (0, 0)),
        pl.BlockSpec((1, _D_HID), lambda i: (0, 0)),
    ],
    out_specs=[
        pl.BlockSpec((_BLK, _D_HID), lambda i: (i, 0)),
        pl.BlockSpec((_BLK, _D_HID), lambda i: (i, 0)),
    ],
    out_shape=[
        jax.ShapeDtypeStruct((_N, _D_HID), jnp.float32),
        jax.ShapeDtypeStruct((_N, _D_HID), jnp.float32),
    ],
)


def _inv_bcast(cnt):
  total = jnp.sum(cnt, axis=0, keepdims=True)
  inv = 1.0 / jnp.maximum(total, 1.0)
  ones = jnp.ones((1, _D_HID), jnp.float32)
  return lax.dot_general(inv, ones, (((0,), (0,)), ((), ())),
                         preferred_element_type=jnp.float32)


def _tc2_body(agg_ref, cnt_ref, r1_ref, h_ref):
  invb = _inv_bcast(cnt_ref[...])
  mean = (agg_ref[0] + agg_ref[1]) * invb
  h_ref[...] = jnp.maximum(mean + r1_ref[...], 0.0)


_tc2 = pl.pallas_call(
    _tc2_body,
    in_specs=[
        pl.BlockSpec((_NC, _N, _D_HID), lambda: (0, 0, 0)),
        pl.BlockSpec((_NW, _N), lambda: (0, 0)),
        pl.BlockSpec((_N, _D_HID), lambda: (0, 0)),
    ],
    out_specs=pl.BlockSpec((_N, _D_HID), lambda: (0, 0)),
    out_shape=jax.ShapeDtypeStruct((_N, _D_HID), jnp.float32),
)


def _tc3_body(agg_ref, cnt_ref, h_ref, wl_ref, wr_ref, bl_ref, out_ref):
  invb = _inv_bcast(cnt_ref[...])
  mean = (agg_ref[0] + agg_ref[1]) * invb
  out_ref[...] = (
      lax.dot_general(mean, wl_ref[...], (((1,), (1,)), ((), ())),
                      preferred_element_type=jnp.float32)
      + lax.dot_general(h_ref[...], wr_ref[...], (((1,), (1,)), ((), ())),
                        preferred_element_type=jnp.float32)
      + bl_ref[...])


_tc3 = pl.pallas_call(
    _tc3_body,
    in_specs=[
        pl.BlockSpec((_NC, _N, _D_HID), lambda: (0, 0, 0)),
        pl.BlockSpec((_NW, _N), lambda: (0, 0)),
        pl.BlockSpec((_N, _D_HID), lambda: (0, 0)),
        pl.BlockSpec((_D_OUT, _D_HID), lambda: (0, 0)),
        pl.BlockSpec((_D_OUT, _D_HID), lambda: (0, 0)),
        pl.BlockSpec((1, _D_OUT), lambda: (0, 0)),
    ],
    out_specs=pl.BlockSpec((_N, _D_OUT), lambda: (0, 0)),
    out_shape=jax.ShapeDtypeStruct((_N, _D_OUT), jnp.float32),
)


def kernel(x, edge_index, Wl1, bl1, Wr1, Wl2, bl2, Wr2):
  src = edge_index[0]
  dst = edge_index[1]

  z128 = jnp.zeros((_RPT, _D_HID), jnp.float32)
  zc = jnp.zeros((1, _N), jnp.float32)

  p1, r1 = _tc1(x, Wl1, Wr1, bl1.reshape(1, -1))
  agg1, cnt = _sc_agg(_D_HID, True)(p1, src, dst, z128, zc)
  cnt2 = cnt.reshape(_NW, _N)
  h = _tc2(agg1, cnt2, r1)
  agg2 = _sc_agg(_D_HID, False)(h, src, dst, z128)
  return _tc3(agg2, cnt2, h, Wl2, Wr2, bl2.reshape(1, -1))
